# baseline (speedup 1.0000x reference)
import jax
import jax.numpy as jnp
from jax import lax
from jax.experimental import pallas as pl
from jax.experimental.pallas import tpu as pltpu
from jax.experimental.pallas import tpu_sc as plsc

_D = 129
_N = 100000
_B = 16384
_NW = 32
_RPW = (_D - 1) // _NW
_L = 16
_OCHUNK = 4096
_SEG = _B // _NW


def _gather_body(idx_hbm, tabt_hbm, last_hbm, outt_hbm,
                 idx_v, row_v, ob0, ob1, last_v, sem_w, sem_g):
    wid = lax.axis_index("s") * 2 + lax.axis_index("c")
    pltpu.sync_copy(idx_hbm, idx_v)
    zeros16 = jnp.zeros((_L,), jnp.int32)

    seg = wid * _SEG
    last_copies = [
        pltpu.async_copy(
            last_hbm.at[idx_v.at[0, pl.ds(seg + j * 128, 128)]],
            last_v.at[0, pl.ds(j * 128, 128)], sem_g)
        for j in range(_SEG // 128)
    ]

    bufs = (ob0, ob1)
    pending = []

    def gather_chunk(buf, cbase):
        @plsc.parallel_loop(0, _OCHUNK // _L, unroll=8)
        def _(b):
            idx16 = idx_v[0, pl.ds(cbase + b * _L, _L)]
            buf[0, pl.ds(b * _L, _L)] = plsc.load_gather(
                row_v, [zeros16, idx16])

    for t in range(_RPW):
        r = wid * _RPW + t
        pltpu.sync_copy(tabt_hbm.at[pl.ds(r, 1)], row_v)
        for cc in range(_B // _OCHUNK):
            buf = bufs[cc % 2]
            if len(pending) >= 2:
                pending.pop(0).wait()
            gather_chunk(buf, cc * _OCHUNK)
            pending.append(pltpu.async_copy(
                buf, outt_hbm.at[pl.ds(r, 1), pl.ds(cc * _OCHUNK, _OCHUNK)],
                sem_w))
    while pending:
        pending.pop(0).wait()
    for c in last_copies:
        c.wait()
    pltpu.sync_copy(last_v, outt_hbm.at[pl.ds(_D - 1, 1), pl.ds(seg, _SEG)])


def kernel(batch, table):
    idx2d = batch.astype(jnp.int32).reshape(1, _B)
    tabt = table.T
    tab_last = table[:, _D - 1]
    run = pl.kernel(
        _gather_body,
        out_type=jax.ShapeDtypeStruct((_D, _B), jnp.float32),
        mesh=plsc.VectorSubcoreMesh(core_axis_name="c", subcore_axis_name="s"),
        scratch_types=[
            pltpu.VMEM((1, _B), jnp.int32),
            pltpu.VMEM((1, _N), jnp.float32),
            pltpu.VMEM((1, _OCHUNK), jnp.float32),
            pltpu.VMEM((1, _OCHUNK), jnp.float32),
            pltpu.VMEM((1, _SEG), jnp.float32),
            pltpu.SemaphoreType.DMA,
            pltpu.SemaphoreType.DMA,
        ],
        compiler_params=pltpu.CompilerParams(
            needs_layout_passes=False, skip_device_barrier=True),
    )
    outt = run(idx2d, tabt, tab_last)
    return outt.T


# speedup vs baseline: 1.0081x; 1.0081x over previous
import jax
import jax.numpy as jnp
from jax import lax
from jax.experimental import pallas as pl
from jax.experimental.pallas import tpu as pltpu
from jax.experimental.pallas import tpu_sc as plsc

_D = 129
_N = 100000
_B = 16384
_NW = 32
_RPW = (_D - 1) // _NW
_L = 16
_OCHUNK = 4096
_SEG = _B // _NW


def _gather_body(idx_hbm, tabt_hbm, last_hbm, outt_hbm,
                 idx_v, row_v, ob0, ob1, last_v, sem_w, sem_g, sem_s):
    wid = lax.axis_index("s") * 2 + lax.axis_index("c")
    row0_stage = pltpu.async_copy(
        tabt_hbm.at[pl.ds(wid * _RPW, 1)], row_v, sem_s)
    pltpu.sync_copy(idx_hbm, idx_v)
    zeros16 = jnp.zeros((_L,), jnp.int32)

    seg = wid * _SEG
    last_copies = [
        pltpu.async_copy(
            last_hbm.at[idx_v.at[0, pl.ds(seg + j * 128, 128)]],
            last_v.at[0, pl.ds(j * 128, 128)], sem_g)
        for j in range(_SEG // 128)
    ]

    bufs = (ob0, ob1)
    pending = []

    def gather_chunk(buf, cbase):
        @plsc.parallel_loop(0, _OCHUNK // _L, unroll=8)
        def _(b):
            idx16 = idx_v[0, pl.ds(cbase + b * _L, _L)]
            buf[0, pl.ds(b * _L, _L)] = plsc.load_gather(
                row_v, [zeros16, idx16])

    for t in range(_RPW):
        r = wid * _RPW + t
        if t == 0:
            row0_stage.wait()
        else:
            pltpu.sync_copy(tabt_hbm.at[pl.ds(r, 1)], row_v)
        for cc in range(_B // _OCHUNK):
            buf = bufs[cc % 2]
            if len(pending) >= 2:
                pending.pop(0).wait()
            gather_chunk(buf, cc * _OCHUNK)
            pending.append(pltpu.async_copy(
                buf, outt_hbm.at[pl.ds(r, 1), pl.ds(cc * _OCHUNK, _OCHUNK)],
                sem_w))
    while pending:
        pending.pop(0).wait()
    for c in last_copies:
        c.wait()
    pltpu.sync_copy(last_v, outt_hbm.at[pl.ds(_D - 1, 1), pl.ds(seg, _SEG)])


def kernel(batch, table):
    idx2d = batch.astype(jnp.int32).reshape(1, _B)
    tabt = table.T
    tab_last = table[:, _D - 1]
    run = pl.kernel(
        _gather_body,
        out_type=jax.ShapeDtypeStruct((_D, _B), jnp.float32),
        mesh=plsc.VectorSubcoreMesh(core_axis_name="c", subcore_axis_name="s"),
        scratch_types=[
            pltpu.VMEM((1, _B), jnp.int32),
            pltpu.VMEM((1, _N), jnp.float32),
            pltpu.VMEM((1, _OCHUNK), jnp.float32),
            pltpu.VMEM((1, _OCHUNK), jnp.float32),
            pltpu.VMEM((1, _SEG), jnp.float32),
            pltpu.SemaphoreType.DMA,
            pltpu.SemaphoreType.DMA,
            pltpu.SemaphoreType.DMA,
        ],
        compiler_params=pltpu.CompilerParams(needs_layout_passes=False),
    )
    outt = run(idx2d, tabt, tab_last)
    return outt.T
